# baseline (speedup 1.0000x reference)
import functools

import jax
import jax.numpy as jnp
from jax import lax
from jax.experimental import pallas as pl
from jax.experimental.pallas import tpu as pltpu
from jax.experimental.pallas import tpu_sc as plsc

L = 16
RU = 4
NC = 2
NS = 16
BLK = 128
D_TC = 256
CH_M = 128
CH_B = 64


def _col_segs(width):
    segs, off = [], 0
    while off < width:
        seg = min(128, width - off)
        segs.append((off, seg))
        off += seg
    return segs


def _make_sc(total, d, b):
    d_sc = d - D_TC
    ng_m = d_sc // L
    ng_b = D_TC // L

    mesh = plsc.VectorSubcoreMesh(
        core_axis_name="c", subcore_axis_name="s",
        num_cores=NC, num_subcores=NS)

    @functools.partial(
        pl.kernel,
        out_type=(
            jax.ShapeDtypeStruct((NC, b, d_sc), jnp.float32),
            jax.ShapeDtypeStruct((NC, b, D_TC), jnp.float32),
        ),
        mesh=mesh,
        scratch_types=[
            pltpu.VMEM((2, CH_M, d_sc), jnp.float32),
            pltpu.VMEM((2, CH_B, D_TC), jnp.float32),
            pltpu.VMEM((L,), jnp.int32),
            pltpu.VMEM((1, d), jnp.float32),
            pltpu.VMEM_SHARED((NS, d), jnp.float32),
            pltpu.SemaphoreType.DMA,
            pltpu.SemaphoreType.DMA,
        ],
        compiler_params=pltpu.CompilerParams(needs_layout_passes=False),
    )
    def sc_seg(x_hbm, sizes_hbm, out1, out2, bufm, bufb, sz_v, out_v, board,
               sem0, sem1):
        core = lax.axis_index("c")
        sub = lax.axis_index("s")
        bag = sub

        pltpu.sync_copy(sizes_hbm, sz_v)
        sizes_vec = sz_v[...]
        starts_vec = lax.cumsum(sizes_vec, axis=0) - sizes_vec
        lane = lax.iota(jnp.int32, L)
        sel = lane == bag
        start = jnp.max(jnp.where(sel, starts_vec, 0))
        size = jnp.max(jnp.where(sel, sizes_vec, 0))
        end = start + size

        neg_inf = jnp.full((L,), -jnp.inf, dtype=jnp.float32)
        sems = (sem0, sem1)

        def seg_reduce(buf, ch, colw, col0, ngrp, lo, hi, accs, ru=RU):
            a_lo = 8 * (lo // 8)
            a_hi = 8 * ((hi + 7) // 8)
            span = a_hi - a_lo
            n_ch = (span + ch - 1) // ch
            hi_base = jnp.maximum(a_hi - ch, 0)

            def chunk_base(i):
                return pl.multiple_of(jnp.minimum(a_lo + i * ch, hi_base), 8)

            def chunk_slice(i):
                return x_hbm.at[pl.ds(chunk_base(i), ch), pl.ds(col0, colw)]

            def start_dma(i, k):
                pltpu.async_copy(chunk_slice(i), buf.at[k], sems[k])

            def wait_dma(i, k):
                pltpu.make_async_copy(chunk_slice(i), buf.at[k], sems[k]).wait()

            def process_chunk(i, k, accs):
                base = chunk_base(i)
                head = jnp.clip(lo - base, 0, ch)
                tail = jnp.clip(base + ch - hi, 0, ch)

                def blank_head(r, _):
                    for j in range(ngrp):
                        buf[k, r, pl.ds(j * L, L)] = neg_inf
                    return 0

                def blank_tail(r, _):
                    for j in range(ngrp):
                        buf[k, ch - 1 - r, pl.ds(j * L, L)] = neg_inf
                    return 0

                lax.fori_loop(0, head, blank_head, 0)
                lax.fori_loop(0, tail, blank_tail, 0)

                def body(r4, accs):
                    r = r4 * ru
                    for rr in range(ru):
                        accs = tuple(
                            jnp.maximum(accs[j],
                                        buf[k, r + rr, pl.ds(j * L, L)])
                            for j in range(ngrp))
                    return accs
                return lax.fori_loop(0, ch // ru, body, accs)

            n_pair = n_ch // 2

            @pl.when(n_pair > 0)
            def _():
                start_dma(0, 0)

            def pair_body(p, accs):
                i0 = 2 * p
                start_dma(i0 + 1, 1)
                wait_dma(i0, 0)
                accs = process_chunk(i0, 0, accs)

                @pl.when(i0 + 2 < n_pair * 2)
                def _():
                    start_dma(i0 + 2, 0)

                wait_dma(i0 + 1, 1)
                return process_chunk(i0 + 1, 1, accs)

            accs = lax.fori_loop(0, n_pair, pair_body, accs)

            def odd_fn(accs):
                pltpu.async_copy(chunk_slice(n_pair * 2), buf.at[0],
                                 sem0).wait()
                return process_chunk(n_pair * 2, 0, accs)

            return lax.cond(n_ch % 2 == 1, odd_fn, lambda a: a, accs)

        mid = start + size // 2
        m_lo = jnp.where(core == 0, start, mid)
        m_hi = jnp.where(core == 0, mid, end)
        accs = seg_reduce(bufm, CH_M, d_sc, D_TC, ng_m, m_lo, m_hi,
                          (neg_inf,) * ng_m)
        for j in range(ng_m):
            out_v[0, pl.ds(j * L, L)] = accs[j]

        b_lo = jnp.where(core == 0, start,
                         jnp.maximum(start, BLK * (end // BLK)))
        b_hi = jnp.where(core == 0,
                         jnp.minimum(end, BLK * ((start + BLK - 1) // BLK)),
                         end)
        baccs = seg_reduce(bufb, CH_B, D_TC, 0, ng_b, b_lo, b_hi,
                           (neg_inf,) * ng_b, ru=1)
        for j in range(ng_b):
            out_v[0, pl.ds(d_sc + j * L, L)] = baccs[j]

        pltpu.sync_copy(out_v, board.at[pl.ds(bag, 1)])
        plsc.subcore_barrier()

        jobs = []
        for rt in range(b // 8):
            for c0, cw in _col_segs(d_sc):
                jobs.append((out1, 8 * rt, 0, c0, cw))
            for c0, cw in _col_segs(D_TC):
                jobs.append((out2, 8 * rt, d_sc, c0, cw))
        for w, (dst, r0, boff, c0, cw) in enumerate(jobs):
            @pl.when(sub == w)
            def _(dst=dst, r0=r0, boff=boff, c0=c0, cw=cw):
                pltpu.sync_copy(
                    board.at[pl.ds(r0, 8), pl.ds(boff + c0, cw)],
                    dst.at[core, pl.ds(r0, 8), pl.ds(c0, cw)])

    return sc_seg


def _make_tc(total, d, b):
    step_rows = 4096
    sub = step_rows // BLK
    nstep = total // step_rows

    def tc_body(starts_ref, ends_ref, x_ref, o_ref):
        minus_inf = jnp.float32(-jnp.inf)
        kb = pl.program_id(0)

        @pl.when(kb == 0)
        def _():
            o_ref[...] = jnp.full((b, D_TC), minus_inf, jnp.float32)

        starts = starts_ref[...]
        ends = ends_ref[...]
        acc = o_ref[...]
        for i in range(sub):
            lo = kb * step_rows + i * BLK
            full = (starts <= lo) & (ends >= lo + BLK)
            bm = jnp.max(x_ref[pl.ds(i * BLK, BLK), :], axis=0, keepdims=True)
            acc = jnp.maximum(acc, jnp.where(full, bm, minus_inf))
        o_ref[...] = acc

    return pl.pallas_call(
        tc_body,
        grid=(nstep,),
        in_specs=[
            pl.BlockSpec((b, 1), lambda kb: (0, 0)),
            pl.BlockSpec((b, 1), lambda kb: (0, 0)),
            pl.BlockSpec((step_rows, D_TC), lambda kb: (kb, 0)),
        ],
        out_specs=pl.BlockSpec((b, D_TC), lambda kb: (0, 0)),
        out_shape=jax.ShapeDtypeStruct((b, D_TC), jnp.float32),
    )


def kernel(inter_pre, bags_size):
    total, d = inter_pre.shape
    b = bags_size.shape[0]
    assert b == L and d > D_TC and (d - D_TC) % (2 * L) == 0
    assert total % BLK == 0 and total >= CH_M
    sizes = bags_size.astype(jnp.int32)
    ends = jnp.cumsum(sizes)
    starts = (ends - sizes).reshape(b, 1)
    ends = ends.reshape(b, 1)

    sc_seg = _make_sc(total, d, b)
    tc_blk = _make_tc(total, d, b)
    out1, out2 = sc_seg(inter_pre, sizes)
    t1 = tc_blk(starts, ends, inter_pre)

    s1 = jnp.maximum(out1[0], out1[1])
    s2 = jnp.maximum(out2[0], out2[1])
    return jnp.concatenate([jnp.maximum(t1, s2), s1], axis=1)


# speedup vs baseline: 1.1193x; 1.1193x over previous
import functools

import jax
import jax.numpy as jnp
from jax import lax
from jax.experimental import pallas as pl
from jax.experimental.pallas import tpu as pltpu
from jax.experimental.pallas import tpu_sc as plsc

L = 16
RU = 4
NC = 2
NS = 16
BLK = 128
D_TC = 384
CH_M = 256
CH_B = 64


def _col_segs(width):
    segs, off = [], 0
    while off < width:
        seg = min(128, width - off)
        segs.append((off, seg))
        off += seg
    return segs


def _make_sc(total, d, b):
    d_sc = d - D_TC
    ng_m = d_sc // L
    ng_b = D_TC // L

    mesh = plsc.VectorSubcoreMesh(
        core_axis_name="c", subcore_axis_name="s",
        num_cores=NC, num_subcores=NS)

    @functools.partial(
        pl.kernel,
        out_type=(
            jax.ShapeDtypeStruct((NC, b, d_sc), jnp.float32),
            jax.ShapeDtypeStruct((NC, b, D_TC), jnp.float32),
        ),
        mesh=mesh,
        scratch_types=[
            pltpu.VMEM((2, CH_M, d_sc), jnp.float32),
            pltpu.VMEM((2, CH_B, D_TC), jnp.float32),
            pltpu.VMEM((L,), jnp.int32),
            pltpu.VMEM((1, d), jnp.float32),
            pltpu.VMEM_SHARED((NS, d), jnp.float32),
            pltpu.SemaphoreType.DMA,
            pltpu.SemaphoreType.DMA,
        ],
        compiler_params=pltpu.CompilerParams(needs_layout_passes=False),
    )
    def sc_seg(x_hbm, sizes_hbm, out1, out2, bufm, bufb, sz_v, out_v, board,
               sem0, sem1):
        core = lax.axis_index("c")
        sub = lax.axis_index("s")
        bag = sub

        pltpu.sync_copy(sizes_hbm, sz_v)
        sizes_vec = sz_v[...]
        starts_vec = lax.cumsum(sizes_vec, axis=0) - sizes_vec
        lane = lax.iota(jnp.int32, L)
        sel = lane == bag
        start = jnp.max(jnp.where(sel, starts_vec, 0))
        size = jnp.max(jnp.where(sel, sizes_vec, 0))
        end = start + size

        neg_inf = jnp.full((L,), -jnp.inf, dtype=jnp.float32)
        sems = (sem0, sem1)

        def seg_reduce(buf, ch, colw, col0, ngrp, lo, hi, accs, ru=RU):
            a_lo = 8 * (lo // 8)
            a_hi = 8 * ((hi + 7) // 8)
            span = a_hi - a_lo
            n_ch = (span + ch - 1) // ch
            hi_base = jnp.maximum(a_hi - ch, 0)

            def chunk_base(i):
                return pl.multiple_of(jnp.minimum(a_lo + i * ch, hi_base), 8)

            def chunk_slice(i):
                return x_hbm.at[pl.ds(chunk_base(i), ch), pl.ds(col0, colw)]

            def start_dma(i, k):
                pltpu.async_copy(chunk_slice(i), buf.at[k], sems[k])

            def wait_dma(i, k):
                pltpu.make_async_copy(chunk_slice(i), buf.at[k], sems[k]).wait()

            def process_chunk(i, k, accs):
                base = chunk_base(i)
                head = jnp.clip(lo - base, 0, ch)
                tail = jnp.clip(base + ch - hi, 0, ch)

                def blank_head(r, _):
                    for j in range(ngrp):
                        buf[k, r, pl.ds(j * L, L)] = neg_inf
                    return 0

                def blank_tail(r, _):
                    for j in range(ngrp):
                        buf[k, ch - 1 - r, pl.ds(j * L, L)] = neg_inf
                    return 0

                lax.fori_loop(0, head, blank_head, 0)
                lax.fori_loop(0, tail, blank_tail, 0)

                def body(r4, accs):
                    r = r4 * ru
                    for rr in range(ru):
                        accs = tuple(
                            jnp.maximum(accs[j],
                                        buf[k, r + rr, pl.ds(j * L, L)])
                            for j in range(ngrp))
                    return accs
                return lax.fori_loop(0, ch // ru, body, accs)

            n_pair = n_ch // 2

            @pl.when(n_pair > 0)
            def _():
                start_dma(0, 0)

            def pair_body(p, accs):
                i0 = 2 * p
                start_dma(i0 + 1, 1)
                wait_dma(i0, 0)
                accs = process_chunk(i0, 0, accs)

                @pl.when(i0 + 2 < n_pair * 2)
                def _():
                    start_dma(i0 + 2, 0)

                wait_dma(i0 + 1, 1)
                return process_chunk(i0 + 1, 1, accs)

            accs = lax.fori_loop(0, n_pair, pair_body, accs)

            def odd_fn(accs):
                pltpu.async_copy(chunk_slice(n_pair * 2), buf.at[0],
                                 sem0).wait()
                return process_chunk(n_pair * 2, 0, accs)

            return lax.cond(n_ch % 2 == 1, odd_fn, lambda a: a, accs)

        mid = start + size // 2
        m_lo = jnp.where(core == 0, start, mid)
        m_hi = jnp.where(core == 0, mid, end)
        accs = seg_reduce(bufm, CH_M, d_sc, D_TC, ng_m, m_lo, m_hi,
                          (neg_inf,) * ng_m)
        for j in range(ng_m):
            out_v[0, pl.ds(j * L, L)] = accs[j]

        b_lo = jnp.where(core == 0, start,
                         jnp.maximum(start, BLK * (end // BLK)))
        b_hi = jnp.where(core == 0,
                         jnp.minimum(end, BLK * ((start + BLK - 1) // BLK)),
                         end)
        baccs = seg_reduce(bufb, CH_B, D_TC, 0, ng_b, b_lo, b_hi,
                           (neg_inf,) * ng_b, ru=1)
        for j in range(ng_b):
            out_v[0, pl.ds(d_sc + j * L, L)] = baccs[j]

        pltpu.sync_copy(out_v, board.at[pl.ds(bag, 1)])
        plsc.subcore_barrier()

        jobs = []
        for rt in range(b // 8):
            for c0, cw in _col_segs(d_sc):
                jobs.append((out1, 8 * rt, 0, c0, cw))
            for c0, cw in _col_segs(D_TC):
                jobs.append((out2, 8 * rt, d_sc, c0, cw))
        for w, (dst, r0, boff, c0, cw) in enumerate(jobs):
            @pl.when(sub == w)
            def _(dst=dst, r0=r0, boff=boff, c0=c0, cw=cw):
                pltpu.sync_copy(
                    board.at[pl.ds(r0, 8), pl.ds(boff + c0, cw)],
                    dst.at[core, pl.ds(r0, 8), pl.ds(c0, cw)])

    return sc_seg


def _make_tc(total, d, b):
    step_rows = 4096
    sub = step_rows // BLK
    nstep = total // step_rows

    def tc_body(starts_ref, ends_ref, x_ref, o_ref):
        minus_inf = jnp.float32(-jnp.inf)
        kb = pl.program_id(0)

        @pl.when(kb == 0)
        def _():
            o_ref[...] = jnp.full((b, D_TC), minus_inf, jnp.float32)

        starts = starts_ref[...]
        ends = ends_ref[...]
        acc = o_ref[...]
        for i in range(sub):
            lo = kb * step_rows + i * BLK
            full = (starts <= lo) & (ends >= lo + BLK)
            bm = jnp.max(x_ref[pl.ds(i * BLK, BLK), :], axis=0, keepdims=True)
            acc = jnp.maximum(acc, jnp.where(full, bm, minus_inf))
        o_ref[...] = acc

    return pl.pallas_call(
        tc_body,
        grid=(nstep,),
        in_specs=[
            pl.BlockSpec((b, 1), lambda kb: (0, 0)),
            pl.BlockSpec((b, 1), lambda kb: (0, 0)),
            pl.BlockSpec((step_rows, D_TC), lambda kb: (kb, 0)),
        ],
        out_specs=pl.BlockSpec((b, D_TC), lambda kb: (0, 0)),
        out_shape=jax.ShapeDtypeStruct((b, D_TC), jnp.float32),
    )


def kernel(inter_pre, bags_size):
    total, d = inter_pre.shape
    b = bags_size.shape[0]
    assert b == L and d > D_TC and (d - D_TC) % (2 * L) == 0
    assert total % BLK == 0 and total >= CH_M
    sizes = bags_size.astype(jnp.int32)
    ends = jnp.cumsum(sizes)
    starts = (ends - sizes).reshape(b, 1)
    ends = ends.reshape(b, 1)

    sc_seg = _make_sc(total, d, b)
    tc_blk = _make_tc(total, d, b)
    out1, out2 = sc_seg(inter_pre, sizes)
    t1 = tc_blk(starts, ends, inter_pre)

    s1 = jnp.maximum(out1[0], out1[1])
    s2 = jnp.maximum(out2[0], out2[1])
    return jnp.concatenate([jnp.maximum(t1, s2), s1], axis=1)
